# baseline (speedup 1.0000x reference)
import functools

import jax
import jax.numpy as jnp
from jax import lax
from jax.experimental import pallas as pl
from jax.experimental.pallas import tpu as pltpu
from jax.experimental.pallas import tpu_sc as plsc

_VOCAB = 1000000
_D = 64
_NC = 2
_NS = 16
_NW = _NC * _NS
_CH = 128
_SCALE = 8.0


def _tec_body(n_chunks, idx_hbm, table_hbm, out_hbm, idx_v, rows_v, sem):
    wid = lax.axis_index("s") * _NC + lax.axis_index("c")
    pltpu.sync_copy(idx_hbm.at[wid], idx_v)
    base = wid * (n_chunks * _CH)

    def chunk(j):
        pltpu.async_copy(table_hbm.at[idx_v.at[j]], rows_v, sem).wait()

        def row(i):
            for c in range(_D // 16):
                sl = pl.ds(c * 16, 16)
                rows_v[i, sl] = rows_v[i, sl] * _SCALE

        pl.loop(0, _CH)(row)
        pltpu.sync_copy(rows_v, out_hbm.at[pl.ds(base + j * _CH, _CH)])

    pl.loop(0, n_chunks)(chunk)


@jax.jit
def kernel(x, table):
    b, h = x.shape
    n = b * h
    assert n % (_NW * _CH) == 0
    n_chunks = n // (_NW * _CH)
    idx = x.reshape(_NW, n_chunks, _CH).astype(jnp.int32)

    mesh = plsc.VectorSubcoreMesh(core_axis_name="c", subcore_axis_name="s")
    out = pl.kernel(
        functools.partial(_tec_body, n_chunks),
        out_type=jax.ShapeDtypeStruct((n, _D), jnp.float32),
        mesh=mesh,
        scratch_types=[
            pltpu.VMEM((n_chunks, _CH), jnp.int32),
            pltpu.VMEM((_CH, _D), jnp.float32),
            pltpu.SemaphoreType.DMA,
        ],
        compiler_params=pltpu.CompilerParams(use_tc_tiling_on_sc=False),
    )(idx, table)
    return out.reshape(b, h, _D)


# speedup vs baseline: 1.2080x; 1.2080x over previous
import functools

import jax
import jax.numpy as jnp
from jax import lax
from jax.experimental import pallas as pl
from jax.experimental.pallas import tpu as pltpu
from jax.experimental.pallas import tpu_sc as plsc

_D = 64
_NC = 2
_NS = 16
_NW = _NC * _NS
_CH = 128
_NBUF = 4
_SCALE = 8.0


def _tec_body(n_chunks, idx_hbm, table_hbm, out_hbm, idx_v, gbufs, sbufs,
              gsems, ssems):
    wid = lax.axis_index("s") * _NC + lax.axis_index("c")
    pltpu.sync_copy(idx_hbm.at[wid], idx_v)
    base = wid * (n_chunks * _CH)

    def gather_start(j, b):
        pltpu.async_copy(table_hbm.at[idx_v.at[j]], gbufs[b], gsems[b])

    def gather_wait(j, b):
        pltpu.make_async_copy(table_hbm.at[idx_v.at[j]], gbufs[b],
                              gsems[b]).wait()

    def scatter_start(j, b):
        pltpu.async_copy(sbufs[b], out_hbm.at[pl.ds(base + j * _CH, _CH)],
                         ssems[b])

    def scatter_wait(j, b):
        pltpu.make_async_copy(sbufs[b], out_hbm.at[pl.ds(base + j * _CH, _CH)],
                              ssems[b]).wait()

    for b in range(_NBUF):
        gather_start(b, b)

    def outer(g):
        for b in range(_NBUF):
            j = g + b
            gather_wait(j, b)

            @pl.when(j >= _NBUF)
            def _():
                scatter_wait(j - _NBUF, b)

            gbuf, sbuf = gbufs[b], sbufs[b]

            def row(i):
                for c in range(_D // 16):
                    sl = pl.ds(c * 16, 16)
                    sbuf[i, sl] = gbuf[i, sl] * _SCALE

            plsc.parallel_loop(0, _CH, 1, unroll=4)(row)

            scatter_start(j, b)

            @pl.when(j + _NBUF < n_chunks)
            def _():
                gather_start(j + _NBUF, b)

    pl.loop(0, n_chunks, step=_NBUF)(outer)

    for b in range(_NBUF):
        scatter_wait(n_chunks - _NBUF + b, b)


@jax.jit
def kernel(x, table):
    b, h = x.shape
    n = b * h
    assert n % (_NW * _CH * _NBUF) == 0
    n_chunks = n // (_NW * _CH)
    idx = x.reshape(_NW, n_chunks, _CH).astype(jnp.int32)

    mesh = plsc.VectorSubcoreMesh(core_axis_name="c", subcore_axis_name="s")
    out = pl.kernel(
        functools.partial(_tec_body, n_chunks),
        out_type=jax.ShapeDtypeStruct((n, _D), jnp.float32),
        mesh=mesh,
        scratch_types=[
            pltpu.VMEM((n_chunks, _CH), jnp.int32),
            [pltpu.VMEM((_CH, _D), jnp.float32) for _ in range(_NBUF)],
            [pltpu.VMEM((_CH, _D), jnp.float32) for _ in range(_NBUF)],
            [pltpu.SemaphoreType.DMA for _ in range(_NBUF)],
            [pltpu.SemaphoreType.DMA for _ in range(_NBUF)],
        ],
        compiler_params=pltpu.CompilerParams(use_tc_tiling_on_sc=False),
    )(idx, table)
    return out.reshape(b, h, _D)


# speedup vs baseline: 1.2106x; 1.0021x over previous
import functools

import jax
import jax.numpy as jnp
from jax import lax
from jax.experimental import pallas as pl
from jax.experimental.pallas import tpu as pltpu
from jax.experimental.pallas import tpu_sc as plsc

_D = 64
_NC = 2
_NS = 16
_NW = _NC * _NS
_K = 16
_CH = _K * 16
_NBUF = 2
_SCALE = 8.0


def _tec_body(n_chunks, idx_hbm, table_hbm, out_hbm, idx_v, gbufs, sbufs,
              gsems, ssems):
    wid = lax.axis_index("s") * _NC + lax.axis_index("c")
    pltpu.sync_copy(idx_hbm.at[wid], idx_v)
    base = wid * (n_chunks * _CH)

    def gather_start(j, b):
        for k in range(_K):
            ids = idx_v[j, pl.ds(k * 16, 16)]
            pltpu.async_copy(table_hbm.at[ids], gbufs[b].at[pl.ds(k * 16, 16)],
                             gsems[b])

    def gather_wait(b):
        pltpu.make_async_copy(table_hbm.at[pl.ds(0, _CH)], gbufs[b],
                              gsems[b]).wait()

    def scatter_start(j, b):
        pltpu.async_copy(sbufs[b], out_hbm.at[pl.ds(base + j * _CH, _CH)],
                         ssems[b])

    def scatter_wait(j, b):
        pltpu.make_async_copy(sbufs[b], out_hbm.at[pl.ds(base + j * _CH, _CH)],
                              ssems[b]).wait()

    for b in range(_NBUF):
        gather_start(b, b)

    def outer(g):
        for b in range(_NBUF):
            j = g + b
            gather_wait(b)

            @pl.when(j >= _NBUF)
            def _():
                scatter_wait(j - _NBUF, b)

            gbuf, sbuf = gbufs[b], sbufs[b]

            def row(i):
                for c in range(_D // 16):
                    sl = pl.ds(c * 16, 16)
                    sbuf[i, sl] = gbuf[i, sl] * _SCALE

            plsc.parallel_loop(0, _CH, 1, unroll=4)(row)

            scatter_start(j, b)

            @pl.when(j + _NBUF < n_chunks)
            def _():
                gather_start(j + _NBUF, b)

    pl.loop(0, n_chunks, step=_NBUF)(outer)

    for b in range(_NBUF):
        scatter_wait(n_chunks - _NBUF + b, b)


@jax.jit
def kernel(x, table):
    b, h = x.shape
    n = b * h
    assert n % (_NW * _CH * _NBUF) == 0
    n_chunks = n // (_NW * _CH)
    idx = x.reshape(_NW, n_chunks, _CH).astype(jnp.int32)

    mesh = plsc.VectorSubcoreMesh(core_axis_name="c", subcore_axis_name="s")
    out = pl.kernel(
        functools.partial(_tec_body, n_chunks),
        out_type=jax.ShapeDtypeStruct((n, _D), jnp.float32),
        mesh=mesh,
        scratch_types=[
            pltpu.VMEM((n_chunks, _CH), jnp.int32),
            [pltpu.VMEM((_CH, _D), jnp.float32) for _ in range(_NBUF)],
            [pltpu.VMEM((_CH, _D), jnp.float32) for _ in range(_NBUF)],
            [pltpu.SemaphoreType.DMA for _ in range(_NBUF)],
            [pltpu.SemaphoreType.DMA for _ in range(_NBUF)],
        ],
        compiler_params=pltpu.CompilerParams(use_tc_tiling_on_sc=False),
    )(idx, table)
    return out.reshape(b, h, _D)
